# baseline (speedup 1.0000x reference)
import jax
import jax.numpy as jnp
from jax.experimental import pallas as pl
from jax.experimental.pallas import tpu as pltpu


def _fused_kernel(a_ref, x_ref, w0_ref, w1_ref, fc1w_ref, gfb_ref,
                  fc2w_ref, fc2b_ref, fc3w_ref, fc3b_ref,
                  fc4w_ref, fc4b_ref, out_ref):
    a = a_ref[...]
    x = x_ref[...]
    y = jnp.dot(x, w0_ref[...], preferred_element_type=jnp.float32)
    h = jax.nn.relu(jnp.dot(a, y, preferred_element_type=jnp.float32))
    y = jnp.dot(h, w1_ref[...], preferred_element_type=jnp.float32)
    h = jax.nn.relu(jnp.dot(a, y, preferred_element_type=jnp.float32))
    n = h.shape[0]
    avg = jnp.sum(h, axis=0, keepdims=True) / n
    var = jnp.sum(jnp.square(h), axis=0, keepdims=True) / n - jnp.square(avg)
    var = jnp.where(var <= 0.0, jnp.float32(1e-06), var)
    std = jnp.sqrt(var)
    mx = jnp.max(h, axis=0, keepdims=True)
    mn = jnp.min(h, axis=0, keepdims=True)
    z = jnp.concatenate((avg, std, mx, mn), axis=1)
    z = jax.nn.relu(jnp.dot(z, fc1w_ref[...],
                            preferred_element_type=jnp.float32) + gfb_ref[...])
    z = jax.nn.relu(jnp.dot(z, fc2w_ref[...],
                            preferred_element_type=jnp.float32) + fc2b_ref[...])
    z = jax.nn.relu(jnp.dot(z, fc3w_ref[...],
                            preferred_element_type=jnp.float32) + fc3b_ref[...])
    z = jnp.dot(z, fc4w_ref[...],
                preferred_element_type=jnp.float32) + fc4b_ref[...]
    col = jax.lax.broadcasted_iota(jnp.int32, z.shape, 1)
    half = z.shape[1] // 2
    out_ref[...] = jnp.where(col >= half, jnp.square(z), z)


def kernel(x_0, x_1, x_2, x_3, x_4, n0_to_0, n1_to_1, n2_to_2, n3_to_3,
           n4_to_4, n0_to_1, n0_to_2, n0_to_3, n0_to_4, n1_to_2, n1_to_3,
           n1_to_4, n2_to_3, n2_to_4, n3_to_4, global_feature,
           W0_0, W0_1, W0_2, W0_3, W0_4, W1_0, W1_1, W1_2, W1_3, W1_4,
           fc1_w, fc1_b, fc2_w, fc2_b, fc3_w, fc3_b, fc4_w, fc4_b):
    d = x_0.shape[1]
    gfb = (global_feature[:, :4] @ fc1_w[4 * d:] + fc1_b)[None, 0]
    out = pl.pallas_call(
        _fused_kernel,
        out_shape=jax.ShapeDtypeStruct((1, 2), jnp.float32),
    )(n0_to_0, x_0, W0_0, W1_0, fc1_w[:4 * d], gfb,
      fc2_w, fc2_b[None, :], fc3_w, fc3_b[None, :],
      fc4_w, fc4_b[None, :])
    return out


# speedup vs baseline: 1.5559x; 1.5559x over previous
import jax
import jax.numpy as jnp
from jax.experimental import pallas as pl
from jax.experimental.pallas import tpu as pltpu

_N = 2048
_D = 128
_NCHUNK = 8
_CH = _N // _NCHUNK


def _fused_kernel(a_hbm, x_ref, w0_ref, w1_ref, gf_ref,
                  fc1w_ref, fc1b_ref, fc2w_ref, fc2b_ref,
                  fc3w_ref, fc3b_ref, fc4w_ref, fc4b_ref, out_ref,
                  a_vmem, h_vmem, sems):
    for c in range(_NCHUNK):
        pltpu.make_async_copy(
            a_hbm.at[pl.ds(c * _CH, _CH), :],
            a_vmem.at[pl.ds(c * _CH, _CH), :],
            sems.at[c],
        ).start()
    y = jnp.dot(x_ref[...], w0_ref[...], preferred_element_type=jnp.float32)
    for c in range(_NCHUNK):
        pltpu.make_async_copy(
            a_hbm.at[pl.ds(c * _CH, _CH), :],
            a_vmem.at[pl.ds(c * _CH, _CH), :],
            sems.at[c],
        ).wait()
        h_vmem[pl.ds(c * _CH, _CH), :] = jax.nn.relu(
            jnp.dot(a_vmem[pl.ds(c * _CH, _CH), :], y,
                    preferred_element_type=jnp.float32))
    y = jnp.dot(h_vmem[...], w1_ref[...], preferred_element_type=jnp.float32)
    h = jax.nn.relu(jnp.dot(a_vmem[...], y,
                            preferred_element_type=jnp.float32))
    avg = jnp.sum(h, axis=0, keepdims=True) / _N
    var = jnp.sum(jnp.square(h), axis=0, keepdims=True) / _N - jnp.square(avg)
    var = jnp.where(var <= 0.0, jnp.float32(1e-06), var)
    std = jnp.sqrt(var)
    mx = jnp.max(h, axis=0, keepdims=True)
    mn = jnp.min(h, axis=0, keepdims=True)
    z = jnp.concatenate((avg, std, mx, mn), axis=1)
    z = (jnp.dot(z, fc1w_ref[:4 * _D, :], preferred_element_type=jnp.float32)
         + jnp.dot(gf_ref[...], fc1w_ref[4 * _D:, :],
                   preferred_element_type=jnp.float32)
         + fc1b_ref[...])
    z = jax.nn.relu(z)
    z = jax.nn.relu(jnp.dot(z, fc2w_ref[...],
                            preferred_element_type=jnp.float32) + fc2b_ref[...])
    z = jax.nn.relu(jnp.dot(z, fc3w_ref[...],
                            preferred_element_type=jnp.float32) + fc3b_ref[...])
    z = jnp.dot(z, fc4w_ref[...],
                preferred_element_type=jnp.float32) + fc4b_ref[...]
    col = jax.lax.broadcasted_iota(jnp.int32, z.shape, 1)
    half = z.shape[1] // 2
    out_ref[...] = jnp.where(col >= half, jnp.square(z), z)


def kernel(x_0, x_1, x_2, x_3, x_4, n0_to_0, n1_to_1, n2_to_2, n3_to_3,
           n4_to_4, n0_to_1, n0_to_2, n0_to_3, n0_to_4, n1_to_2, n1_to_3,
           n1_to_4, n2_to_3, n2_to_4, n3_to_4, global_feature,
           W0_0, W0_1, W0_2, W0_3, W0_4, W1_0, W1_1, W1_2, W1_3, W1_4,
           fc1_w, fc1_b, fc2_w, fc2_b, fc3_w, fc3_b, fc4_w, fc4_b):
    out = pl.pallas_call(
        _fused_kernel,
        out_shape=jax.ShapeDtypeStruct((1, 2), jnp.float32),
        in_specs=[pl.BlockSpec(memory_space=pltpu.MemorySpace.HBM)] +
                 [pl.BlockSpec(memory_space=pltpu.MemorySpace.VMEM)] * 12,
        scratch_shapes=[
            pltpu.MemorySpace.VMEM((_N, _N), jnp.float32),
            pltpu.MemorySpace.VMEM((_N, _D), jnp.float32),
            pltpu.SemaphoreType.DMA((_NCHUNK,)),
        ],
    )(n0_to_0, x_0, W0_0, W1_0, global_feature,
      fc1_w, fc1_b[None, :], fc2_w, fc2_b[None, :],
      fc3_w, fc3_b[None, :], fc4_w, fc4_b[None, :])
    return out
